# baseline (speedup 1.0000x reference)
import functools

import jax
import jax.numpy as jnp
from jax import lax
from jax.experimental import pallas as pl
from jax.experimental.pallas import tpu as pltpu
from jax.experimental.pallas import tpu_sc as plsc

BB = 320000
FF = 144
DD = 128

NC = 2
NS = 16
NW = NC * NS

_SC_MESH = dict(core_axis_name="c", subcore_axis_name="s")
_SC_PARAMS = pltpu.CompilerParams(needs_layout_passes=False)


def _wid():
    return lax.axis_index("s") * NC + lax.axis_index("c")


CSHIFT = 13
CHUNK = 1 << CSHIFT
NCHUNK = 40
NCP = 48
CAP = 8960
AK = 112
A_NBLK = 5
PB = 80
G_PER_W = BB // NW
P_NBLK = G_PER_W // PB


def _hist_body(src_hbm, counts_hbm, srcb, hist, counts_v, sem):
    del sem
    wid = _wid()
    i16 = lax.iota(jnp.int32, 16)
    zeros = jnp.zeros((16,), jnp.int32)
    ones = jnp.ones((16,), jnp.int32)
    for i in range(16 * NCP // 16):
        hist[pl.ds(i * 16, 16)] = zeros

    stripe0 = wid * G_PER_W

    def blk(bi, carry):
        pltpu.sync_copy(src_hbm.at[pl.ds(stripe0 + bi * 2000, 2000)], srcb)

        def vec(v, c2):
            p = srcb[pl.ds(v * 16, 16)]
            c = p >> CSHIFT
            plsc.addupdate_scatter(hist, [i16 * NCP + c], ones)
            return c2

        lax.fori_loop(0, 125, vec, 0)
        return carry

    lax.fori_loop(0, 5, blk, 0)

    for cb in range(NCP // 16):
        acc = jnp.zeros((16,), jnp.int32)
        for l in range(16):
            acc = acc + hist[pl.ds(l * NCP + cb * 16, 16)]
        counts_v[pl.ds(cb * 16, 16)] = acc
    pltpu.sync_copy(counts_v, counts_hbm.at[pl.ds(wid * NCP, NCP)])


@jax.jit
def _sc_hist(src_idx):
    return pl.kernel(
        _hist_body,
        out_type=jax.ShapeDtypeStruct((NW * NCP,), jnp.int32),
        mesh=plsc.VectorSubcoreMesh(**_SC_MESH),
        compiler_params=_SC_PARAMS,
        scratch_types=[
            pltpu.VMEM((2000,), jnp.int32),
            pltpu.VMEM((16 * NCP,), jnp.int32),
            pltpu.VMEM((NCP,), jnp.int32),
            pltpu.SemaphoreType.DMA,
        ],
    )(src_idx)


def _perm_blk_compute(v, base, srcb, neib, cntref, posb, valb, dstvb, neivb,
                      shbuf, i16):
    p = srcb[pl.ds(v * 16, 16)]
    c = p >> CSHIFT
    ck, lane = plsc.sort_key_val(c, i16)
    shbuf[pl.ds(0, 16)] = ck
    shbuf[pl.ds(1, 16)] = ck
    prev = shbuf[pl.ds(0, 16)]
    shbuf[pl.ds(40, 16)] = ck
    nxt = shbuf[pl.ds(41, 16)]
    change = jnp.not_equal(ck, prev)
    start = plsc.cummax(jnp.where(change, i16, jnp.zeros((16,), jnp.int32)))
    rank = i16 - start
    prior = plsc.load_gather(cntref, [ck])
    pos = prior + rank
    plsc.store_scatter(cntref, [ck], pos + 1, mask=jnp.not_equal(ck, nxt))
    ps = plsc.load_gather(srcb, [lane + v * 16])
    pn = plsc.load_gather(neib, [lane + v * 16])
    posb[pl.ds(v * 16, 16)] = pos
    valb[pl.ds(v * 16, 16)] = base + v * 16 + lane
    dstvb[pl.ds(v * 16, 16)] = ps & (CHUNK - 1)
    neivb[pl.ds(v * 16, 16)] = pn


def _perm_body(src_hbm, nei_hbm, counts_hbm, perm_hbm, dst_hbm, neio_hbm,
               countsb, cntref, srcb0, srcb1, neib0, neib1, posb0, posb1,
               valb0, valb1, dstvb0, dstvb1, neivb0, neivb1, shbuf,
               semi0, semi1, semo0, semo1):
    wid = _wid()
    i16 = lax.iota(jnp.int32, 16)
    pltpu.sync_copy(counts_hbm, countsb)
    shbuf[pl.ds(56, 16)] = jnp.full((16,), -1, jnp.int32)

    for cb in range(NCP // 16):
        def acc_body(w, a):
            return a + countsb[pl.ds(w * NCP + cb * 16, 16)]

        pw = lax.fori_loop(0, wid, acc_body, jnp.zeros((16,), jnp.int32))
        cntref[pl.ds(cb * 16, 16)] = (i16 + cb * 16) * CAP + pw

    stripe0 = wid * G_PER_W
    srcb = (srcb0, srcb1)
    neib = (neib0, neib1)
    posb = (posb0, posb1)
    valb = (valb0, valb1)
    dstvb = (dstvb0, dstvb1)
    neivb = (neivb0, neivb1)
    semi = (semi0, semi1)
    semo = (semo0, semo1)

    def issue_in(j, b):
        base = stripe0 + j * PB
        pltpu.async_copy(src_hbm.at[pl.ds(base, PB)], srcb[b], semi[b])
        pltpu.async_copy(nei_hbm.at[pl.ds(base, PB)], neib[b], semi[b])

    def drain_in(b):
        pltpu.make_async_copy(src_hbm.at[pl.ds(0, PB)], srcb[b],
                              semi[b]).wait()
        pltpu.make_async_copy(nei_hbm.at[pl.ds(0, PB)], neib[b],
                              semi[b]).wait()

    def drain_out(b):
        pltpu.make_async_copy(valb[b], perm_hbm.at[posb[b]], semo[b]).wait()
        pltpu.make_async_copy(dstvb[b], dst_hbm.at[posb[b]], semo[b]).wait()
        pltpu.make_async_copy(neivb[b], neio_hbm.at[posb[b]], semo[b]).wait()

    def do_block(j, b, first):
        drain_in(b)
        if not first:
            drain_out(b)
        base = stripe0 + j * PB
        for v in range(PB // 16):
            _perm_blk_compute(v, base, srcb[b], neib[b], cntref, posb[b],
                              valb[b], dstvb[b], neivb[b], shbuf, i16)
        pltpu.async_copy(valb[b], perm_hbm.at[posb[b]], semo[b])
        pltpu.async_copy(dstvb[b], dst_hbm.at[posb[b]], semo[b])
        pltpu.async_copy(neivb[b], neio_hbm.at[posb[b]], semo[b])

    issue_in(0, 0)
    issue_in(1, 1)
    do_block(0, 0, True)
    issue_in(2, 0)
    do_block(1, 1, True)
    issue_in(3, 1)

    def pair(p, carry):
        j = 2 + 2 * p

        def one(b):
            do_block(j + b, b, False)

            @pl.when(j + b + 2 < P_NBLK)
            def _():
                issue_in(j + b + 2, b)

        one(0)
        one(1)
        return carry

    lax.fori_loop(0, (P_NBLK - 2) // 2, pair, 0)
    do_block(P_NBLK - 1, 0, False)
    drain_out(0)
    drain_out(1)


@jax.jit
def _sc_perm(src_idx, nei_idx, counts):
    osh = jax.ShapeDtypeStruct((NCHUNK * CAP,), jnp.int32)
    ib = pltpu.VMEM((PB,), jnp.int32)
    return pl.kernel(
        _perm_body,
        out_type=(osh, osh, osh),
        mesh=plsc.VectorSubcoreMesh(**_SC_MESH),
        compiler_params=_SC_PARAMS,
        scratch_types=[
            pltpu.VMEM((NW * NCP,), jnp.int32),
            pltpu.VMEM((NCP,), jnp.int32),
            ib, ib, ib, ib, ib, ib, ib, ib, ib, ib, ib, ib,
            pltpu.VMEM((72,), jnp.int32),
            pltpu.SemaphoreType.DMA,
            pltpu.SemaphoreType.DMA,
            pltpu.SemaphoreType.DMA,
            pltpu.SemaphoreType.DMA,
        ],
    )(src_idx, nei_idx, counts)


GK = 80
G_NBLK = G_PER_W // GK


def _gather_body(nei_hbm, mess_hbm, out_hbm, idx_v, rows_v, sem):
    base = _wid() * G_PER_W

    def body(j, carry):
        off = base + j * GK
        pltpu.sync_copy(nei_hbm.at[pl.ds(off, GK)], idx_v)
        pltpu.async_copy(mess_hbm.at[idx_v], rows_v, sem).wait()
        pltpu.sync_copy(rows_v, out_hbm.at[pl.ds(off, GK)])
        return carry

    lax.fori_loop(0, G_NBLK, body, 0)


@jax.jit
def _sc_gather(nei_idx, mess):
    return pl.kernel(
        _gather_body,
        out_type=jax.ShapeDtypeStruct((BB, DD), jnp.float32),
        mesh=plsc.VectorSubcoreMesh(**_SC_MESH),
        compiler_params=_SC_PARAMS,
        scratch_types=[
            pltpu.VMEM((GK,), jnp.int32),
            pltpu.VMEM((GK, DD), jnp.float32),
            pltpu.SemaphoreType.DMA,
        ],
    )(nei_idx, mess)


ACC_ROWS = 10240
CPS = NCHUNK // NC


def _lens_from_counts(countsb, lenbuf):
    for cb in range(NCP // 16):
        def acc_body(w, a):
            return a + countsb[pl.ds(w * NCP + cb * 16, 16)]

        lenbuf[pl.ds(cb * 16, 16)] = lax.fori_loop(
            0, NW, acc_body, jnp.zeros((16,), jnp.int32))


def _accum_body(gidx_hbm, dst_hbm, counts_hbm, val_hbm, zero_hbm, out_hbm,
                countsb, lenbuf, gixb0, gixb1, dstraw0, dstraw1, gbuf0, gbuf1,
                dstb0, dstb1, rowsb0, rowsb1, zbuf, acc, semi0, semi1, semg0,
                semg1):
    ca = lax.axis_index("c")
    t = lax.axis_index("s")
    i16 = lax.iota(jnp.int32, 16)

    pltpu.sync_copy(counts_hbm, countsb)
    pltpu.sync_copy(zero_hbm, zbuf)
    _lens_from_counts(countsb, lenbuf)

    gixb = (gixb0, gixb1)
    dstraw = (dstraw0, dstraw1)
    gbuf = (gbuf0, gbuf1)
    dstb = (dstb0, dstb1)
    rowsb = (rowsb0, rowsb1)
    semi = (semi0, semi1)
    semg = (semg0, semg1)

    def issue_in(c, j, b):
        pos0 = c * CAP + t * (A_NBLK * AK) + j * AK
        pltpu.async_copy(gidx_hbm.at[pl.ds(pos0, AK)], gixb[b], semi[b])
        pltpu.async_copy(dst_hbm.at[pl.ds(pos0, AK)], dstraw[b], semi[b])

    def drain_in(b):
        pltpu.make_async_copy(gidx_hbm.at[pl.ds(0, AK)], gixb[b],
                              semi[b]).wait()
        pltpu.make_async_copy(dst_hbm.at[pl.ds(0, AK)], dstraw[b],
                              semi[b]).wait()

    def start_gather(b):
        drain_in(b)
        for v in range(AK // 16):
            g = gixb[b][pl.ds(v * 16, 16)]
            gbuf[b][pl.ds(v * 16, 16)] = jnp.minimum(jnp.maximum(g, 0),
                                                     BB - 1)
        pltpu.async_copy(val_hbm.at[gbuf[b]], rowsb[b], semg[b])

    def finish_block(c, j, b, lim):
        pltpu.make_async_copy(val_hbm.at[gbuf[b]], rowsb[b], semg[b]).wait()
        pos0 = c * CAP + t * (A_NBLK * AK) + j * AK
        for v in range(AK // 16):
            relpos = pos0 + v * 16 + i16
            trash = CHUNK + ((t * 16 + i16) & 127)
            dstb[b][pl.ds(v * 16, 16)] = jnp.where(
                relpos < lim, dstraw[b][pl.ds(v * 16, 16)], trash)
        pltpu.sync_copy(rowsb[b], acc.at[dstb[b]], add=True)

    def chunk(i, carry):
        c = 2 * i + ca
        lv = lenbuf[pl.ds((c >> 4) * 16, 16)]
        len_c = jnp.sum(jnp.where(i16 == (c & 15), lv, 0))
        lim = c * CAP + len_c
        for z in range(4):
            pltpu.sync_copy(zbuf, acc.at[pl.ds(t * 512 + z * 128, 128)])
        plsc.subcore_barrier()
        issue_in(c, 0, 0)
        issue_in(c, 1, 1)
        start_gather(0)
        for j in range(A_NBLK):
            b = j & 1
            if j + 1 < A_NBLK:
                start_gather(1 - b)
            finish_block(c, j, b, lim)
            if j + 2 < A_NBLK:
                issue_in(c, j + 2, b)
        plsc.subcore_barrier()

        @pl.when(c * CHUNK + t * 512 < BB)
        def _flush():
            pltpu.sync_copy(acc.at[pl.ds(t * 512, 512)],
                            out_hbm.at[pl.ds(c * CHUNK + t * 512, 512)])

        plsc.subcore_barrier()
        return carry

    lax.fori_loop(0, CPS, chunk, 0)


@jax.jit
def _sc_accum(gidx, dstloc, counts, val):
    ib = pltpu.VMEM((AK,), jnp.int32)
    rb = pltpu.VMEM((AK, DD), jnp.float32)
    return pl.kernel(
        _accum_body,
        out_type=jax.ShapeDtypeStruct((BB, DD), jnp.float32),
        mesh=plsc.VectorSubcoreMesh(**_SC_MESH),
        compiler_params=_SC_PARAMS,
        scratch_types=[
            pltpu.VMEM((NW * NCP,), jnp.int32),
            pltpu.VMEM((NCP,), jnp.int32),
            ib, ib, ib, ib, ib, ib, ib, ib, rb, rb,
            pltpu.VMEM((128, DD), jnp.float32),
            pltpu.VMEM_SHARED((ACC_ROWS, DD), jnp.float32),
            pltpu.SemaphoreType.DMA,
            pltpu.SemaphoreType.DMA,
            pltpu.SemaphoreType.DMA,
            pltpu.SemaphoreType.DMA,
        ],
    )(gidx, dstloc, counts, val, jnp.zeros((128, DD), jnp.float32))


ROWS_A = 2560
ROWS_B = 2560


def _a_body(hk_ref, mk_ref, wr1_ref, wr2_ref, br_ref, rm_ref):
    hk = hk_ref[...]
    mk = mk_ref[...]
    acc = (jnp.dot(hk, wr1_ref[...], preferred_element_type=jnp.float32)
           + jnp.dot(mk, wr2_ref[...], preferred_element_type=jnp.float32)
           + br_ref[...])
    rm_ref[...] = jax.nn.sigmoid(acc) * mk


def _dense_rm(h_ki, mess_ki, Wr_w, Wr_b):
    wr1 = Wr_w[:, :FF].T
    wr2 = Wr_w[:, FF:].T
    br = Wr_b.reshape(1, DD)
    nblk = BB // ROWS_A
    return pl.pallas_call(
        _a_body,
        grid=(nblk,),
        in_specs=[
            pl.BlockSpec((ROWS_A, FF), lambda i: (i, 0)),
            pl.BlockSpec((ROWS_A, DD), lambda i: (i, 0)),
            pl.BlockSpec((FF, DD), lambda i: (0, 0)),
            pl.BlockSpec((DD, DD), lambda i: (0, 0)),
            pl.BlockSpec((1, DD), lambda i: (0, 0)),
        ],
        out_specs=pl.BlockSpec((ROWS_A, DD), lambda i: (i, 0)),
        out_shape=jax.ShapeDtypeStruct((BB, DD), jnp.float32),
    )(h_ki, mess_ki, wr1, wr2, br)


def _b_body(h_ref, s_ref, r_ref, wz1_ref, wz2_ref, bz_ref, ww_ref, bw_ref,
            uw_ref, out_ref):
    h = h_ref[...]
    s = s_ref[...]
    r = r_ref[...]
    z = jax.nn.sigmoid(
        jnp.dot(h, wz1_ref[...], preferred_element_type=jnp.float32)
        + jnp.dot(s, wz2_ref[...], preferred_element_type=jnp.float32)
        + bz_ref[...])
    m = jnp.tanh(jnp.dot(h, ww_ref[...], preferred_element_type=jnp.float32)
                 + bw_ref[...]
                 + jnp.dot(r, uw_ref[...], preferred_element_type=jnp.float32))
    out_ref[...] = (1.0 - z) * s + z * m


def _dense_out(h_ij, s_ij, r_ij, Wz_w, Wz_b, U_w, W_w, W_b):
    wz1 = Wz_w[:, :FF].T
    wz2 = Wz_w[:, FF:].T
    bz = Wz_b.reshape(1, DD)
    ww = W_w.T
    bw = W_b.reshape(1, DD)
    uw = U_w.T
    nblk = BB // ROWS_B
    return pl.pallas_call(
        _b_body,
        grid=(nblk,),
        in_specs=[
            pl.BlockSpec((ROWS_B, FF), lambda i: (i, 0)),
            pl.BlockSpec((ROWS_B, DD), lambda i: (i, 0)),
            pl.BlockSpec((ROWS_B, DD), lambda i: (i, 0)),
            pl.BlockSpec((FF, DD), lambda i: (0, 0)),
            pl.BlockSpec((DD, DD), lambda i: (0, 0)),
            pl.BlockSpec((1, DD), lambda i: (0, 0)),
            pl.BlockSpec((FF, DD), lambda i: (0, 0)),
            pl.BlockSpec((1, DD), lambda i: (0, 0)),
            pl.BlockSpec((DD, DD), lambda i: (0, 0)),
        ],
        out_specs=pl.BlockSpec((ROWS_B, DD), lambda i: (i, 0)),
        out_shape=jax.ShapeDtypeStruct((BB, DD), jnp.float32),
    )(h_ij, s_ij, r_ij, wz1, wz2, bz, ww, bw, uw)


def kernel(h_ij, h_ki, mess, src_idx, nei_idx, Wz_w, Wz_b, Wr_w, Wr_b, U_w,
           W_w, W_b):
    counts = _sc_hist(src_idx)
    perm, dstloc, neio = _sc_perm(src_idx, nei_idx, counts)
    mess_ki = _sc_gather(nei_idx, mess)
    s_ij = _sc_accum(neio, dstloc, counts, mess)
    rm = _dense_rm(h_ki, mess_ki, Wr_w, Wr_b)
    r_ij = _sc_accum(perm, dstloc, counts, rm)
    return _dense_out(h_ij, s_ij, r_ij, Wz_w, Wz_b, U_w, W_w, W_b)
